# baseline (speedup 1.0000x reference)
import functools
import math

import jax
import jax.numpy as jnp
from jax import lax
from jax.experimental import pallas as pl
from jax.experimental.pallas import tpu as pltpu
from jax.experimental.pallas import tpu_sc as plsc

D_MODEL = 64
SCALE = math.sqrt(D_MODEL)

NC = 2
NS = 16
NW = NC * NS

STEP = 128
LANES = 16


def _make_kernel(n_steps):
    mesh = plsc.VectorSubcoreMesh(core_axis_name="c", subcore_axis_name="s")

    @functools.partial(
        pl.kernel,
        mesh=mesh,
        compiler_params=pltpu.CompilerParams(needs_layout_passes=False),
        out_type=jax.ShapeDtypeStruct((NW, n_steps, STEP, D_MODEL), jnp.float32),
        scratch_types=[
            pltpu.VMEM((n_steps, STEP), jnp.int32),
            pltpu.VMEM((n_steps, STEP), jnp.int32),
            pltpu.VMEM((STEP, 2 * D_MODEL), jnp.float32),
            pltpu.VMEM((STEP, D_MODEL), jnp.float32),
            pltpu.SemaphoreType.DMA,
        ],
    )
    def k(idx_hbm, par_hbm, tbl2_hbm, out_hbm, idx_v, par_v, rows_v, outb_v, sem):
        wid = lax.axis_index("s") * NC + lax.axis_index("c")
        pltpu.sync_copy(idx_hbm.at[wid], idx_v)
        pltpu.sync_copy(par_hbm.at[wid], par_v)

        lanes = lax.iota(jnp.int32, LANES)

        def step(j, carry):
            pltpu.async_copy(tbl2_hbm.at[idx_v.at[j]], rows_v, sem).wait()

            jv = jnp.full((LANES,), j, jnp.int32)

            def row(r, c2):
                rv = jnp.full((LANES,), r, jnp.int32)
                pv = plsc.load_gather(par_v, [jv, rv])
                cb = pv * D_MODEL + lanes
                for cc in range(D_MODEL // LANES):
                    v = plsc.load_gather(rows_v, [rv, cb + cc * LANES])
                    outb_v[r, pl.ds(cc * LANES, LANES)] = v * SCALE
                return c2

            lax.fori_loop(0, STEP, row, 0, unroll=4)

            pltpu.sync_copy(outb_v, out_hbm.at[wid, j])
            return carry

        lax.fori_loop(0, n_steps, step, 0)

    return k


def kernel(x, table):
    b, s = x.shape
    total = b * s
    assert total % (NW * STEP) == 0
    n_steps = total // (NW * STEP)
    v, d = table.shape
    tbl2 = table.reshape(v // 2, 2 * d)
    xf = x.reshape(-1).astype(jnp.int32)
    idx2 = (xf >> 1).reshape(NW, n_steps, STEP)
    par = (xf & 1).reshape(NW, n_steps, STEP)
    out = _make_kernel(n_steps)(idx2, par, tbl2)
    return out.reshape(b, s, D_MODEL)
